# Initial kernel scaffold; baseline (speedup 1.0000x reference)
import jax
import jax.numpy as jnp
from jax.experimental import pallas as pl


def kernel(flat_obs, segment_ids, W1, ln1_g, ln1_b, W2, ln2_g, ln2_b, W_ih, W_hh, b):
    raise NotImplementedError("write your pallas kernel here")


# speedup vs baseline: 2.9253x; 2.9253x over previous
import jax
import jax.numpy as jnp
from jax.experimental import pallas as pl
from jax.experimental.pallas import tpu as pltpu

IN_FEAT = 128
FEATURES = 512
HIDDEN = 512
N_TIME = 32
N_PLAYERS = 16
TOTAL = 32768
NUM_SEG = N_TIME * N_PLAYERS

BR = 2048
GRID = TOTAL // BR


def _embed_pool_kernel(obs_ref, seg_ref, w1_ref, g1_ref, bb1_ref,
                       w2_ref, g2_ref, bb2_ref, out_ref):
    i = pl.program_id(0)
    x = obs_ref[...]
    h = jnp.dot(x, w1_ref[...], preferred_element_type=jnp.float32)
    h = jnp.where(h >= 0, h, 0.1 * h)
    mu = jnp.mean(h, axis=1, keepdims=True)
    var = jnp.mean((h - mu) ** 2, axis=1, keepdims=True)
    h = (h - mu) / jnp.sqrt(var + 1e-5) * g1_ref[...] + bb1_ref[...]

    e = jnp.dot(h, w2_ref[...], preferred_element_type=jnp.float32)
    e = jnp.where(e >= 0, e, 0.1 * e)
    mu2 = jnp.mean(e, axis=1, keepdims=True)
    var2 = jnp.mean((e - mu2) ** 2, axis=1, keepdims=True)
    e = (e - mu2) / jnp.sqrt(var2 + 1e-5) * g2_ref[...] + bb2_ref[...]

    seg = seg_ref[0, 0, :]
    onehot = (jax.lax.broadcasted_iota(jnp.int32, (NUM_SEG, BR), 0)
              == seg[None, :]).astype(jnp.float32)
    part = jnp.dot(onehot, e, preferred_element_type=jnp.float32,
                   precision=jax.lax.Precision.HIGHEST)

    @pl.when(i == 0)
    def _init():
        out_ref[...] = part

    @pl.when(i > 0)
    def _acc():
        out_ref[...] += part


def _lstm_kernel(x_ref, wih_ref, whh_ref, b_ref, out_ref, xp_ref):
    xp_ref[...] = jnp.dot(x_ref[...], wih_ref[...],
                          preferred_element_type=jnp.float32) + b_ref[...]
    whh = whh_ref[...]

    def step(t, carry):
        h, c = carry
        gates = xp_ref[pl.ds(t * N_PLAYERS, N_PLAYERS), :]
        gates = gates + jnp.dot(h, whh, preferred_element_type=jnp.float32)
        i_g = jax.nn.sigmoid(gates[:, :HIDDEN])
        f_g = jax.nn.sigmoid(gates[:, HIDDEN:2 * HIDDEN])
        g_g = jnp.tanh(gates[:, 2 * HIDDEN:3 * HIDDEN])
        o_g = jax.nn.sigmoid(gates[:, 3 * HIDDEN:])
        c_new = f_g * c + i_g * g_g
        h_new = o_g * jnp.tanh(c_new)
        out_ref[pl.ds(t * N_PLAYERS, N_PLAYERS), :] = h_new
        return (h_new, c_new)

    h0 = jnp.zeros((N_PLAYERS, HIDDEN), dtype=jnp.float32)
    c0 = jnp.zeros((N_PLAYERS, HIDDEN), dtype=jnp.float32)
    jax.lax.fori_loop(0, N_TIME, step, (h0, c0))


def kernel(flat_obs, segment_ids, W1, ln1_g, ln1_b, W2, ln2_g, ln2_b,
           W_ih, W_hh, b):
    seg3 = segment_ids.astype(jnp.int32).reshape(GRID, 1, BR)
    pooled = pl.pallas_call(
        _embed_pool_kernel,
        grid=(GRID,),
        in_specs=[
            pl.BlockSpec((BR, IN_FEAT), lambda i: (i, 0)),
            pl.BlockSpec((1, 1, BR), lambda i: (i, 0, 0)),
            pl.BlockSpec((IN_FEAT, FEATURES // 2), lambda i: (0, 0)),
            pl.BlockSpec((1, FEATURES // 2), lambda i: (0, 0)),
            pl.BlockSpec((1, FEATURES // 2), lambda i: (0, 0)),
            pl.BlockSpec((FEATURES // 2, FEATURES), lambda i: (0, 0)),
            pl.BlockSpec((1, FEATURES), lambda i: (0, 0)),
            pl.BlockSpec((1, FEATURES), lambda i: (0, 0)),
        ],
        out_specs=pl.BlockSpec((NUM_SEG, FEATURES), lambda i: (0, 0)),
        out_shape=jax.ShapeDtypeStruct((NUM_SEG, FEATURES), jnp.float32),
    )(flat_obs, seg3, W1, ln1_g.reshape(1, -1), ln1_b.reshape(1, -1),
      W2, ln2_g.reshape(1, -1), ln2_b.reshape(1, -1))

    hs = pl.pallas_call(
        _lstm_kernel,
        in_specs=[
            pl.BlockSpec((NUM_SEG, FEATURES), lambda: (0, 0)),
            pl.BlockSpec((FEATURES, 4 * HIDDEN), lambda: (0, 0)),
            pl.BlockSpec((HIDDEN, 4 * HIDDEN), lambda: (0, 0)),
            pl.BlockSpec((1, 4 * HIDDEN), lambda: (0, 0)),
        ],
        out_specs=pl.BlockSpec((N_TIME * N_PLAYERS, HIDDEN), lambda: (0, 0)),
        out_shape=jax.ShapeDtypeStruct((N_TIME * N_PLAYERS, HIDDEN),
                                       jnp.float32),
        scratch_shapes=[pltpu.VMEM((N_TIME * N_PLAYERS, 4 * HIDDEN),
                                   jnp.float32)],
    )(pooled, W_ih, W_hh, b.reshape(1, -1))

    return hs.reshape(N_TIME, N_PLAYERS, HIDDEN)


# speedup vs baseline: 4.6285x; 1.5822x over previous
import jax
import jax.numpy as jnp
from jax.experimental import pallas as pl
from jax.experimental.pallas import tpu as pltpu

IN_FEAT = 128
FEATURES = 512
HIDDEN = 512
N_TIME = 32
N_PLAYERS = 16
TOTAL = 32768
NUM_SEG = N_TIME * N_PLAYERS

BR = 2048
GRID = TOTAL // BR


def _embed_pool_kernel(obs_ref, seg_ref, w1_ref, g1_ref, bb1_ref,
                       w2_ref, g2_ref, bb2_ref, out_ref):
    i = pl.program_id(0)
    x = obs_ref[...]
    h = jnp.dot(x, w1_ref[...], preferred_element_type=jnp.float32)
    h = jnp.where(h >= 0, h, 0.1 * h)
    mu = jnp.mean(h, axis=1, keepdims=True)
    var = jnp.mean((h - mu) ** 2, axis=1, keepdims=True)
    h = (h - mu) / jnp.sqrt(var + 1e-5) * g1_ref[...] + bb1_ref[...]

    e = jnp.dot(h, w2_ref[...], preferred_element_type=jnp.float32)
    e = jnp.where(e >= 0, e, 0.1 * e)
    mu2 = jnp.mean(e, axis=1, keepdims=True)
    var2 = jnp.mean((e - mu2) ** 2, axis=1, keepdims=True)
    e = (e - mu2) / jnp.sqrt(var2 + 1e-5) * g2_ref[...] + bb2_ref[...]

    seg = seg_ref[0, 0, :]
    onehot = (jax.lax.broadcasted_iota(jnp.int32, (NUM_SEG, BR), 0)
              == seg[None, :]).astype(jnp.bfloat16)
    hi = e.astype(jnp.bfloat16)
    lo = (e - hi.astype(jnp.float32)).astype(jnp.bfloat16)
    part = (jnp.dot(onehot, hi, preferred_element_type=jnp.float32)
            + jnp.dot(onehot, lo, preferred_element_type=jnp.float32))

    @pl.when(i == 0)
    def _init():
        out_ref[...] = part

    @pl.when(i > 0)
    def _acc():
        out_ref[...] += part


def _lstm_kernel(x_ref, wih_ref, whh_ref, b_ref, out_ref, xp_ref):
    xp_ref[...] = jnp.dot(x_ref[...], wih_ref[...],
                          preferred_element_type=jnp.float32) + b_ref[...]
    whh = whh_ref[...]

    def step(t, carry):
        h, c = carry
        gates = xp_ref[pl.ds(t * N_PLAYERS, N_PLAYERS), :]
        gates = gates + jnp.dot(h, whh, preferred_element_type=jnp.float32)
        i_g = jax.nn.sigmoid(gates[:, :HIDDEN])
        f_g = jax.nn.sigmoid(gates[:, HIDDEN:2 * HIDDEN])
        g_g = jnp.tanh(gates[:, 2 * HIDDEN:3 * HIDDEN])
        o_g = jax.nn.sigmoid(gates[:, 3 * HIDDEN:])
        c_new = f_g * c + i_g * g_g
        h_new = o_g * jnp.tanh(c_new)
        out_ref[pl.ds(t * N_PLAYERS, N_PLAYERS), :] = h_new
        return (h_new, c_new)

    h0 = jnp.zeros((N_PLAYERS, HIDDEN), dtype=jnp.float32)
    c0 = jnp.zeros((N_PLAYERS, HIDDEN), dtype=jnp.float32)
    jax.lax.fori_loop(0, N_TIME, step, (h0, c0))


def kernel(flat_obs, segment_ids, W1, ln1_g, ln1_b, W2, ln2_g, ln2_b,
           W_ih, W_hh, b):
    seg3 = segment_ids.astype(jnp.int32).reshape(GRID, 1, BR)
    pooled = pl.pallas_call(
        _embed_pool_kernel,
        grid=(GRID,),
        in_specs=[
            pl.BlockSpec((BR, IN_FEAT), lambda i: (i, 0)),
            pl.BlockSpec((1, 1, BR), lambda i: (i, 0, 0)),
            pl.BlockSpec((IN_FEAT, FEATURES // 2), lambda i: (0, 0)),
            pl.BlockSpec((1, FEATURES // 2), lambda i: (0, 0)),
            pl.BlockSpec((1, FEATURES // 2), lambda i: (0, 0)),
            pl.BlockSpec((FEATURES // 2, FEATURES), lambda i: (0, 0)),
            pl.BlockSpec((1, FEATURES), lambda i: (0, 0)),
            pl.BlockSpec((1, FEATURES), lambda i: (0, 0)),
        ],
        out_specs=pl.BlockSpec((NUM_SEG, FEATURES), lambda i: (0, 0)),
        out_shape=jax.ShapeDtypeStruct((NUM_SEG, FEATURES), jnp.float32),
    )(flat_obs, seg3, W1, ln1_g.reshape(1, -1), ln1_b.reshape(1, -1),
      W2, ln2_g.reshape(1, -1), ln2_b.reshape(1, -1))

    hs = pl.pallas_call(
        _lstm_kernel,
        in_specs=[
            pl.BlockSpec((NUM_SEG, FEATURES), lambda: (0, 0)),
            pl.BlockSpec((FEATURES, 4 * HIDDEN), lambda: (0, 0)),
            pl.BlockSpec((HIDDEN, 4 * HIDDEN), lambda: (0, 0)),
            pl.BlockSpec((1, 4 * HIDDEN), lambda: (0, 0)),
        ],
        out_specs=pl.BlockSpec((N_TIME * N_PLAYERS, HIDDEN), lambda: (0, 0)),
        out_shape=jax.ShapeDtypeStruct((N_TIME * N_PLAYERS, HIDDEN),
                                       jnp.float32),
        scratch_shapes=[pltpu.VMEM((N_TIME * N_PLAYERS, 4 * HIDDEN),
                                   jnp.float32)],
    )(pooled, W_ih, W_hh, b.reshape(1, -1))

    return hs.reshape(N_TIME, N_PLAYERS, HIDDEN)


# speedup vs baseline: 4.7842x; 1.0337x over previous
import jax
import jax.numpy as jnp
from jax.experimental import pallas as pl
from jax.experimental.pallas import tpu as pltpu

IN_FEAT = 128
FEATURES = 512
HIDDEN = 512
N_TIME = 32
N_PLAYERS = 16
TOTAL = 32768
NUM_SEG = N_TIME * N_PLAYERS

BR = 4096
GRID = TOTAL // BR


def _embed_pool_kernel(obs_ref, seg_ref, w1_ref, g1_ref, bb1_ref,
                       w2_ref, g2_ref, bb2_ref, out_ref):
    i = pl.program_id(0)
    x = obs_ref[...]
    h = jnp.dot(x, w1_ref[...], preferred_element_type=jnp.float32)
    h = jnp.where(h >= 0, h, 0.1 * h)
    mu = jnp.mean(h, axis=1, keepdims=True)
    var = jnp.mean((h - mu) ** 2, axis=1, keepdims=True)
    h = (h - mu) / jnp.sqrt(var + 1e-5) * g1_ref[...] + bb1_ref[...]

    e = jnp.dot(h, w2_ref[...], preferred_element_type=jnp.float32)
    e = jnp.where(e >= 0, e, 0.1 * e)
    mu2 = jnp.mean(e, axis=1, keepdims=True)
    var2 = jnp.mean((e - mu2) ** 2, axis=1, keepdims=True)
    e = (e - mu2) / jnp.sqrt(var2 + 1e-5) * g2_ref[...] + bb2_ref[...]

    seg = seg_ref[0, 0, :]
    onehot = (jax.lax.broadcasted_iota(jnp.int32, (NUM_SEG, BR), 0)
              == seg[None, :]).astype(jnp.bfloat16)
    hi = e.astype(jnp.bfloat16)
    lo = (e - hi.astype(jnp.float32)).astype(jnp.bfloat16)
    part = (jnp.dot(onehot, hi, preferred_element_type=jnp.float32)
            + jnp.dot(onehot, lo, preferred_element_type=jnp.float32))

    @pl.when(i == 0)
    def _init():
        out_ref[...] = part

    @pl.when(i > 0)
    def _acc():
        out_ref[...] += part


def _lstm_kernel(x_ref, wih_ref, whh_ref, b_ref, out_ref, xp_ref):
    xp_ref[...] = jnp.dot(x_ref[...], wih_ref[...],
                          preferred_element_type=jnp.float32) + b_ref[...]
    whh = whh_ref[...]

    def step(t, carry):
        h, c = carry
        gates = xp_ref[pl.ds(t * N_PLAYERS, N_PLAYERS), :]
        gates = gates + jnp.dot(h, whh, preferred_element_type=jnp.float32)
        i_g = jax.nn.sigmoid(gates[:, :HIDDEN])
        f_g = jax.nn.sigmoid(gates[:, HIDDEN:2 * HIDDEN])
        g_g = jnp.tanh(gates[:, 2 * HIDDEN:3 * HIDDEN])
        o_g = jax.nn.sigmoid(gates[:, 3 * HIDDEN:])
        c_new = f_g * c + i_g * g_g
        h_new = o_g * jnp.tanh(c_new)
        out_ref[pl.ds(t * N_PLAYERS, N_PLAYERS), :] = h_new
        return (h_new, c_new)

    h0 = jnp.zeros((N_PLAYERS, HIDDEN), dtype=jnp.float32)
    c0 = jnp.zeros((N_PLAYERS, HIDDEN), dtype=jnp.float32)
    jax.lax.fori_loop(0, N_TIME, step, (h0, c0))


def kernel(flat_obs, segment_ids, W1, ln1_g, ln1_b, W2, ln2_g, ln2_b,
           W_ih, W_hh, b):
    seg3 = segment_ids.astype(jnp.int32).reshape(GRID, 1, BR)
    pooled = pl.pallas_call(
        _embed_pool_kernel,
        grid=(GRID,),
        in_specs=[
            pl.BlockSpec((BR, IN_FEAT), lambda i: (i, 0)),
            pl.BlockSpec((1, 1, BR), lambda i: (i, 0, 0)),
            pl.BlockSpec((IN_FEAT, FEATURES // 2), lambda i: (0, 0)),
            pl.BlockSpec((1, FEATURES // 2), lambda i: (0, 0)),
            pl.BlockSpec((1, FEATURES // 2), lambda i: (0, 0)),
            pl.BlockSpec((FEATURES // 2, FEATURES), lambda i: (0, 0)),
            pl.BlockSpec((1, FEATURES), lambda i: (0, 0)),
            pl.BlockSpec((1, FEATURES), lambda i: (0, 0)),
        ],
        out_specs=pl.BlockSpec((NUM_SEG, FEATURES), lambda i: (0, 0)),
        out_shape=jax.ShapeDtypeStruct((NUM_SEG, FEATURES), jnp.float32),
    )(flat_obs, seg3, W1, ln1_g.reshape(1, -1), ln1_b.reshape(1, -1),
      W2, ln2_g.reshape(1, -1), ln2_b.reshape(1, -1))

    hs = pl.pallas_call(
        _lstm_kernel,
        in_specs=[
            pl.BlockSpec((NUM_SEG, FEATURES), lambda: (0, 0)),
            pl.BlockSpec((FEATURES, 4 * HIDDEN), lambda: (0, 0)),
            pl.BlockSpec((HIDDEN, 4 * HIDDEN), lambda: (0, 0)),
            pl.BlockSpec((1, 4 * HIDDEN), lambda: (0, 0)),
        ],
        out_specs=pl.BlockSpec((N_TIME * N_PLAYERS, HIDDEN), lambda: (0, 0)),
        out_shape=jax.ShapeDtypeStruct((N_TIME * N_PLAYERS, HIDDEN),
                                       jnp.float32),
        scratch_shapes=[pltpu.VMEM((N_TIME * N_PLAYERS, 4 * HIDDEN),
                                   jnp.float32)],
    )(pooled, W_ih, W_hh, b.reshape(1, -1))

    return hs.reshape(N_TIME, N_PLAYERS, HIDDEN)


# speedup vs baseline: 4.7935x; 1.0019x over previous
import jax
import jax.numpy as jnp
from jax.experimental import pallas as pl
from jax.experimental.pallas import tpu as pltpu

IN_FEAT = 128
FEATURES = 512
HIDDEN = 512
N_TIME = 32
N_PLAYERS = 16
TOTAL = 32768
NUM_SEG = N_TIME * N_PLAYERS

BR = 4096
GRID = TOTAL // BR


def _embed_pool_kernel(obs_ref, seg_ref, w1_ref, g1_ref, bb1_ref,
                       w2_ref, g2_ref, bb2_ref, out_ref):
    i = pl.program_id(0)
    x = obs_ref[...]
    h = jnp.dot(x, w1_ref[...], preferred_element_type=jnp.float32)
    h = jnp.where(h >= 0, h, 0.1 * h)
    mu = jnp.mean(h, axis=1, keepdims=True)
    var = jnp.mean((h - mu) ** 2, axis=1, keepdims=True)
    h = (h - mu) / jnp.sqrt(var + 1e-5) * g1_ref[...] + bb1_ref[...]

    e = jnp.dot(h, w2_ref[...], preferred_element_type=jnp.float32)
    e = jnp.where(e >= 0, e, 0.1 * e)
    mu2 = jnp.mean(e, axis=1, keepdims=True)
    var2 = jnp.mean((e - mu2) ** 2, axis=1, keepdims=True)
    e = (e - mu2) / jnp.sqrt(var2 + 1e-5) * g2_ref[...] + bb2_ref[...]

    seg = seg_ref[0, 0, :]
    onehot = (jax.lax.broadcasted_iota(jnp.int32, (NUM_SEG, BR), 0)
              == seg[None, :]).astype(jnp.bfloat16)
    hi = e.astype(jnp.bfloat16)
    lo = (e - hi.astype(jnp.float32)).astype(jnp.bfloat16)
    part = (jnp.dot(onehot, hi, preferred_element_type=jnp.float32)
            + jnp.dot(onehot, lo, preferred_element_type=jnp.float32))

    @pl.when(i == 0)
    def _init():
        out_ref[...] = part

    @pl.when(i > 0)
    def _acc():
        out_ref[...] += part


def _lstm_kernel(x_ref, wih_ref, whh_ref, b_ref, out_ref, xp_ref):
    xp_ref[...] = jnp.dot(x_ref[...], wih_ref[...],
                          preferred_element_type=jnp.float32) + b_ref[...]
    whh = whh_ref[...]

    w_i = whh[:, :HIDDEN]
    w_f = whh[:, HIDDEN:2 * HIDDEN]
    w_g = whh[:, 2 * HIDDEN:3 * HIDDEN]
    w_o = whh[:, 3 * HIDDEN:]

    def step(t, carry):
        h, c = carry
        xp = xp_ref[pl.ds(t * N_PLAYERS, N_PLAYERS), :]
        i_g = jax.nn.sigmoid(
            xp[:, :HIDDEN]
            + jnp.dot(h, w_i, preferred_element_type=jnp.float32))
        f_g = jax.nn.sigmoid(
            xp[:, HIDDEN:2 * HIDDEN]
            + jnp.dot(h, w_f, preferred_element_type=jnp.float32))
        g_g = jnp.tanh(
            xp[:, 2 * HIDDEN:3 * HIDDEN]
            + jnp.dot(h, w_g, preferred_element_type=jnp.float32))
        o_g = jax.nn.sigmoid(
            xp[:, 3 * HIDDEN:]
            + jnp.dot(h, w_o, preferred_element_type=jnp.float32))
        c_new = f_g * c + i_g * g_g
        h_new = o_g * jnp.tanh(c_new)
        out_ref[pl.ds(t * N_PLAYERS, N_PLAYERS), :] = h_new
        return (h_new, c_new)

    h0 = jnp.zeros((N_PLAYERS, HIDDEN), dtype=jnp.float32)
    c0 = jnp.zeros((N_PLAYERS, HIDDEN), dtype=jnp.float32)
    jax.lax.fori_loop(0, N_TIME, step, (h0, c0))


def kernel(flat_obs, segment_ids, W1, ln1_g, ln1_b, W2, ln2_g, ln2_b,
           W_ih, W_hh, b):
    seg3 = segment_ids.astype(jnp.int32).reshape(GRID, 1, BR)
    pooled = pl.pallas_call(
        _embed_pool_kernel,
        grid=(GRID,),
        in_specs=[
            pl.BlockSpec((BR, IN_FEAT), lambda i: (i, 0)),
            pl.BlockSpec((1, 1, BR), lambda i: (i, 0, 0)),
            pl.BlockSpec((IN_FEAT, FEATURES // 2), lambda i: (0, 0)),
            pl.BlockSpec((1, FEATURES // 2), lambda i: (0, 0)),
            pl.BlockSpec((1, FEATURES // 2), lambda i: (0, 0)),
            pl.BlockSpec((FEATURES // 2, FEATURES), lambda i: (0, 0)),
            pl.BlockSpec((1, FEATURES), lambda i: (0, 0)),
            pl.BlockSpec((1, FEATURES), lambda i: (0, 0)),
        ],
        out_specs=pl.BlockSpec((NUM_SEG, FEATURES), lambda i: (0, 0)),
        out_shape=jax.ShapeDtypeStruct((NUM_SEG, FEATURES), jnp.float32),
    )(flat_obs, seg3, W1, ln1_g.reshape(1, -1), ln1_b.reshape(1, -1),
      W2, ln2_g.reshape(1, -1), ln2_b.reshape(1, -1))

    hs = pl.pallas_call(
        _lstm_kernel,
        in_specs=[
            pl.BlockSpec((NUM_SEG, FEATURES), lambda: (0, 0)),
            pl.BlockSpec((FEATURES, 4 * HIDDEN), lambda: (0, 0)),
            pl.BlockSpec((HIDDEN, 4 * HIDDEN), lambda: (0, 0)),
            pl.BlockSpec((1, 4 * HIDDEN), lambda: (0, 0)),
        ],
        out_specs=pl.BlockSpec((N_TIME * N_PLAYERS, HIDDEN), lambda: (0, 0)),
        out_shape=jax.ShapeDtypeStruct((N_TIME * N_PLAYERS, HIDDEN),
                                       jnp.float32),
        scratch_shapes=[pltpu.VMEM((N_TIME * N_PLAYERS, 4 * HIDDEN),
                                   jnp.float32)],
    )(pooled, W_ih, W_hh, b.reshape(1, -1))

    return hs.reshape(N_TIME, N_PLAYERS, HIDDEN)
